# Initial kernel scaffold; baseline (speedup 1.0000x reference)
import jax
import jax.numpy as jnp
from jax.experimental import pallas as pl


def kernel(x, edge_index, edge_attr, batch, W_node, b_node, W_edge, b_edge, W_g0, b_g0, W_g1, b_g1, W_g2, b_g2, W_reg, b_reg):
    raise NotImplementedError("write your pallas kernel here")


# speedup vs baseline: 10.7379x; 10.7379x over previous
import functools

import jax
import jax.numpy as jnp
from jax import lax
from jax.experimental import pallas as pl
from jax.experimental.pallas import tpu as pltpu
from jax.experimental.pallas import tpu_sc as plsc

N = 10000
E = 320000
D = 128
B = 8

NC = 2
NS = 16
NW = NC * NS
EPT = E // NW
K = 80
NCHUNK = EPT // K

NT_IO = 10
ROWS_PT = N // NT_IO

_MESH = plsc.VectorSubcoreMesh(
    core_axis_name="c", subcore_axis_name="s", num_cores=NC, num_subcores=NS)
_SC_PARAMS = pltpu.CompilerParams(
    use_tc_tiling_on_sc=False, needs_layout_passes=False)

MM_BLK = 2000


def _mm_body(a_ref, w_ref, b_ref, o_ref):
    o_ref[...] = (
        jnp.dot(a_ref[...], w_ref[...], preferred_element_type=jnp.float32)
        + b_ref[...])


def _mm(a, w, b2):
    n, k = a.shape
    m = w.shape[1]
    grid = n // MM_BLK
    return pl.pallas_call(
        _mm_body,
        grid=(grid,),
        in_specs=[
            pl.BlockSpec((MM_BLK, k), lambda i: (i, 0)),
            pl.BlockSpec((k, m), lambda i: (0, 0)),
            pl.BlockSpec((1, m), lambda i: (0, 0)),
        ],
        out_specs=pl.BlockSpec((MM_BLK, m), lambda i: (i, 0)),
        out_shape=jax.ShapeDtypeStruct((n, m), jnp.float32),
    )(a, w, b2)


def _dis_body(d0_ref, d1_ref, dis_ref, dis2_ref):
    deg = d0_ref[...] + d1_ref[...] + 1.0
    dis = jnp.where(deg > 0, lax.rsqrt(jnp.maximum(deg, 1e-12)), 0.0)
    dis_ref[...] = dis
    dis2_ref[...] = dis * dis


def _dis(d0, d1):
    return pl.pallas_call(
        _dis_body,
        out_shape=(jax.ShapeDtypeStruct((80, 125), jnp.float32),
                   jax.ShapeDtypeStruct((80, 125), jnp.float32)),
    )(d0, d1)


def _comb_mm_body(p0_ref, p1_ref, hw_ref, dis2_ref, b_ref, w_ref, o_ref):
    h = p0_ref[...] + p1_ref[...] + dis2_ref[...] * hw_ref[...] + b_ref[...]
    h = jnp.maximum(h, 0.0)
    o_ref[...] = jnp.dot(h, w_ref[...], preferred_element_type=jnp.float32)


def _comb_mm(p0, p1, hw, dis2, b2, w):
    grid = N // MM_BLK
    return pl.pallas_call(
        _comb_mm_body,
        grid=(grid,),
        in_specs=[
            pl.BlockSpec((MM_BLK, D), lambda i: (i, 0)),
            pl.BlockSpec((MM_BLK, D), lambda i: (i, 0)),
            pl.BlockSpec((MM_BLK, D), lambda i: (i, 0)),
            pl.BlockSpec((MM_BLK, 1), lambda i: (i, 0)),
            pl.BlockSpec((1, D), lambda i: (0, 0)),
            pl.BlockSpec((D, D), lambda i: (0, 0)),
        ],
        out_specs=pl.BlockSpec((MM_BLK, D), lambda i: (i, 0)),
        out_shape=jax.ShapeDtypeStruct((N, D), jnp.float32),
    )(p0, p1, hw, dis2, b2, w)


def _pool_body(p0_ref, p1_ref, hw_ref, dis2_ref, b_ref, wreg_ref, batch_ref,
               zsum_ref, cnt_ref):
    @pl.when(pl.program_id(0) == 0)
    def _():
        zsum_ref[...] = jnp.zeros_like(zsum_ref)
        cnt_ref[...] = jnp.zeros_like(cnt_ref)

    h = p0_ref[...] + p1_ref[...] + dis2_ref[...] * hw_ref[...] + b_ref[...]
    h = jnp.maximum(h, 0.0)
    z = jnp.dot(h, wreg_ref[...], preferred_element_type=jnp.float32)
    gids = lax.broadcasted_iota(jnp.int32, (MM_BLK, B), 1)
    oh = (batch_ref[...] == gids).astype(jnp.float32)
    zsum_ref[...] += lax.dot_general(
        oh, z, (((0,), (0,)), ((), ())), preferred_element_type=jnp.float32)
    cnt_ref[...] += lax.dot_general(
        oh, jnp.ones((MM_BLK, 1), jnp.float32), (((0,), (0,)), ((), ())),
        preferred_element_type=jnp.float32)


def _pool(p0, p1, hw, dis2, b2, wreg, batch2):
    grid = N // MM_BLK
    return pl.pallas_call(
        _pool_body,
        grid=(grid,),
        in_specs=[
            pl.BlockSpec((MM_BLK, D), lambda i: (i, 0)),
            pl.BlockSpec((MM_BLK, D), lambda i: (i, 0)),
            pl.BlockSpec((MM_BLK, D), lambda i: (i, 0)),
            pl.BlockSpec((MM_BLK, 1), lambda i: (i, 0)),
            pl.BlockSpec((1, D), lambda i: (0, 0)),
            pl.BlockSpec((D, 1), lambda i: (0, 0)),
            pl.BlockSpec((MM_BLK, 1), lambda i: (i, 0)),
        ],
        out_specs=(pl.BlockSpec((B, 1), lambda i: (0, 0)),
                   pl.BlockSpec((B, 1), lambda i: (0, 0))),
        out_shape=(jax.ShapeDtypeStruct((B, 1), jnp.float32),
                   jax.ShapeDtypeStruct((B, 1), jnp.float32)),
    )(p0, p1, hw, dis2, b2, wreg, batch2)


def _worker_id():
    return lax.axis_index("s") * NC + lax.axis_index("c")


def _zero_vmem(buf, nrows):
    def body(i, _):
        r = i // (D // 16)
        l = (i % (D // 16)) * 16
        buf[r, pl.ds(l, 16)] = jnp.zeros((16,), jnp.float32)
        return 0
    lax.fori_loop(0, nrows * (D // 16), body, 0)


def _deg_body(ew_hbm, col_hbm, out_hbm, ew_v, col_v, zbuf, deg_sh):
    c = lax.axis_index("c")
    s = lax.axis_index("s")
    wid = _worker_id()
    pltpu.sync_copy(ew_hbm.at[pl.ds(wid * EPT, EPT)], ew_v)
    pltpu.sync_copy(col_hbm.at[wid], col_v)

    def zb(i, _):
        zbuf[pl.ds(i * 16, 16)] = jnp.zeros((16,), jnp.float32)
        return 0
    lax.fori_loop(0, 2000 // 16, zb, 0)

    @pl.when(s < 5)
    def _():
        pltpu.sync_copy(zbuf, deg_sh.at[pl.ds(s * 2000, 2000)])
    plsc.subcore_barrier()

    def chunk(j, _):
        pltpu.sync_copy(ew_v.at[pl.ds(j * K, K)],
                        deg_sh.at[col_v.at[j]], add=True)
        return 0
    lax.fori_loop(0, NCHUNK, chunk, 0)
    plsc.subcore_barrier()

    @pl.when(s < 5)
    def _():
        pltpu.sync_copy(deg_sh.at[pl.ds(s * 2000, 2000)], zbuf)
        pltpu.sync_copy(zbuf, out_hbm.at[pl.ds(c * N + s * 2000, 2000)])


_deg_kernel = functools.partial(
    pl.kernel,
    out_type=jax.ShapeDtypeStruct((NC * N,), jnp.float32),
    mesh=_MESH,
    compiler_params=_SC_PARAMS,
    scratch_types=[
        pltpu.MemorySpace.VMEM((EPT,), jnp.float32),
        pltpu.MemorySpace.VMEM((NCHUNK, K), jnp.int32),
        pltpu.MemorySpace.VMEM((2000,), jnp.float32),
        pltpu.MemorySpace.VMEM_SHARED((N,), jnp.float32),
    ],
)(_deg_body)


def _norm_body(dis_hbm, row_hbm, col_hbm, ew_hbm, out_hbm,
               dis_v, row_v, col_v, ew_v, norm_v):
    wid = _worker_id()
    base = wid * EPT
    pltpu.sync_copy(dis_hbm, dis_v)
    pltpu.sync_copy(row_hbm.at[pl.ds(base, EPT)], row_v)
    pltpu.sync_copy(col_hbm.at[pl.ds(base, EPT)], col_v)
    pltpu.sync_copy(ew_hbm.at[pl.ds(base, EPT)], ew_v)

    def body(i, _):
        off = i * 16
        r16 = row_v[pl.ds(off, 16)]
        c16 = col_v[pl.ds(off, 16)]
        e16 = ew_v[pl.ds(off, 16)]
        dr = plsc.load_gather(dis_v, [r16])
        dc = plsc.load_gather(dis_v, [c16])
        norm_v[pl.ds(off, 16)] = dr * e16 * dc
        return 0
    lax.fori_loop(0, EPT // 16, body, 0)
    pltpu.sync_copy(norm_v, out_hbm.at[pl.ds(base, EPT)])


_norm_kernel = functools.partial(
    pl.kernel,
    out_type=jax.ShapeDtypeStruct((E,), jnp.float32),
    mesh=_MESH,
    compiler_params=_SC_PARAMS,
    scratch_types=[
        pltpu.MemorySpace.VMEM((N,), jnp.float32),
        pltpu.MemorySpace.VMEM((EPT,), jnp.int32),
        pltpu.MemorySpace.VMEM((EPT,), jnp.int32),
        pltpu.MemorySpace.VMEM((EPT,), jnp.float32),
        pltpu.MemorySpace.VMEM((EPT,), jnp.float32),
    ],
)(_norm_body)


def _msg_body(hw_hbm, row_hbm, col_hbm, norm_hbm, out_hbm,
              row_v, col_v, norm_v, gbuf, sem, acc_sh):
    c = lax.axis_index("c")
    s = lax.axis_index("s")
    wid = _worker_id()
    pltpu.sync_copy(row_hbm.at[wid], row_v)
    pltpu.sync_copy(col_hbm.at[wid], col_v)
    pltpu.sync_copy(norm_hbm.at[wid], norm_v)

    _zero_vmem(gbuf, K)

    @pl.when(s < NT_IO)
    def _():
        for t in range(ROWS_PT // K):
            pltpu.sync_copy(
                gbuf, acc_sh.at[pl.ds(s * ROWS_PT + t * K, K), :])
        pltpu.sync_copy(
            gbuf.at[pl.ds(0, ROWS_PT - (ROWS_PT // K) * K), :],
            acc_sh.at[pl.ds(s * ROWS_PT + (ROWS_PT // K) * K,
                            ROWS_PT - (ROWS_PT // K) * K), :])
    plsc.subcore_barrier()

    def chunk(j, _):
        pltpu.async_copy(hw_hbm.at[row_v.at[j]], gbuf, sem).wait()

        def group(g, _):
            nv = norm_v[j, pl.ds(g * 16, 16)]
            for e16 in range(16):
                sc = nv[e16]
                e = g * 16 + e16
                for v in range(D // 16):
                    gbuf[e, pl.ds(v * 16, 16)] = (
                        gbuf[e, pl.ds(v * 16, 16)] * sc)
            return 0
        lax.fori_loop(0, K // 16, group, 0)
        pltpu.sync_copy(gbuf, acc_sh.at[col_v.at[j]], add=True)
        return 0
    lax.fori_loop(0, NCHUNK, chunk, 0)
    plsc.subcore_barrier()

    @pl.when(s < NT_IO)
    def _():
        nfull = ROWS_PT // K
        rem = ROWS_PT - nfull * K
        for t in range(nfull):
            r0 = s * ROWS_PT + t * K
            pltpu.sync_copy(acc_sh.at[pl.ds(r0, K), :], gbuf)
            pltpu.sync_copy(gbuf, out_hbm.at[c, pl.ds(r0, K), :])
        r0 = s * ROWS_PT + nfull * K
        pltpu.sync_copy(acc_sh.at[pl.ds(r0, rem), :], gbuf.at[pl.ds(0, rem), :])
        pltpu.sync_copy(gbuf.at[pl.ds(0, rem), :], out_hbm.at[c, pl.ds(r0, rem), :])


_msg_kernel = functools.partial(
    pl.kernel,
    out_type=jax.ShapeDtypeStruct((NC, N, D), jnp.float32),
    mesh=_MESH,
    compiler_params=_SC_PARAMS,
    scratch_types=[
        pltpu.MemorySpace.VMEM((NCHUNK, K), jnp.int32),
        pltpu.MemorySpace.VMEM((NCHUNK, K), jnp.int32),
        pltpu.MemorySpace.VMEM((NCHUNK, K), jnp.float32),
        pltpu.MemorySpace.VMEM((K, D), jnp.float32),
        pltpu.SemaphoreType.DMA,
        pltpu.MemorySpace.VMEM_SHARED((N, D), jnp.float32),
    ],
)(_msg_body)


def kernel(x, edge_index, edge_attr, batch, W_node, b_node, W_edge, b_edge,
           W_g0, b_g0, W_g1, b_g1, W_g2, b_g2, W_reg, b_reg):
    row = edge_index[0]
    col = edge_index[1]
    row3 = row.reshape(NW, NCHUNK, K)
    col3 = col.reshape(NW, NCHUNK, K)

    wbar = jnp.mean(W_edge, axis=1)
    bbar = jnp.mean(b_edge)
    wk = jnp.kron(jnp.eye(32, dtype=jnp.float32), wbar[:, None])

    h0 = _mm(x, W_node, b_node[None, :])

    attr_r = edge_attr.reshape(E // 32, 128)
    ew_mat = _mm(attr_r, wk, jnp.full((1, 32), bbar, jnp.float32))
    ew = ew_mat.reshape(E)

    deg_parts = _deg_kernel(ew, col3)
    dis_m, dis2_m = _dis(deg_parts[:N].reshape(80, 125),
                         deg_parts[N:].reshape(80, 125))
    dis = dis_m.reshape(N)
    dis2 = dis2_m.reshape(N, 1)

    norm = _norm_kernel(dis, row, col, ew)
    norm3 = norm.reshape(NW, NCHUNK, K)

    hw = _mm(h0, W_g0, jnp.zeros((1, D), jnp.float32))
    parts = _msg_kernel(hw, row3, col3, norm3)
    hw1 = _comb_mm(parts[0], parts[1], hw, dis2, b_g0[None, :], W_g1)
    parts1 = _msg_kernel(hw1, row3, col3, norm3)
    hw2 = _comb_mm(parts1[0], parts1[1], hw1, dis2, b_g1[None, :], W_g2)
    parts2 = _msg_kernel(hw2, row3, col3, norm3)

    zsum, cnt = _pool(parts2[0], parts2[1], hw2, dis2, b_g2[None, :],
                      W_reg, batch.reshape(N, 1))
    return zsum / jnp.maximum(cnt, 1.0) + b_reg


# speedup vs baseline: 14.4092x; 1.3419x over previous
import functools

import jax
import jax.numpy as jnp
from jax import lax
from jax.experimental import pallas as pl
from jax.experimental.pallas import tpu as pltpu
from jax.experimental.pallas import tpu_sc as plsc

N = 10000
E = 320000
D = 128
B = 8

NC = 2
NS = 16
NW = NC * NS
EPT = E // NW
K = 80
NCHUNK = EPT // K

NT_IO = 10
ROWS_PT = N // NT_IO

_MESH = plsc.VectorSubcoreMesh(
    core_axis_name="c", subcore_axis_name="s", num_cores=NC, num_subcores=NS)
_SC_PARAMS = pltpu.CompilerParams(
    use_tc_tiling_on_sc=False, needs_layout_passes=False)

MM_BLK = 2000


def _mm_body(a_ref, w_ref, b_ref, o_ref):
    o_ref[...] = (
        jnp.dot(a_ref[...], w_ref[...], preferred_element_type=jnp.float32)
        + b_ref[...])


def _mm(a, w, b2):
    n, k = a.shape
    m = w.shape[1]
    grid = n // MM_BLK
    return pl.pallas_call(
        _mm_body,
        grid=(grid,),
        in_specs=[
            pl.BlockSpec((MM_BLK, k), lambda i: (i, 0)),
            pl.BlockSpec((k, m), lambda i: (0, 0)),
            pl.BlockSpec((1, m), lambda i: (0, 0)),
        ],
        out_specs=pl.BlockSpec((MM_BLK, m), lambda i: (i, 0)),
        out_shape=jax.ShapeDtypeStruct((n, m), jnp.float32),
    )(a, w, b2)


def _dis_body(d0_ref, d1_ref, dis_ref, dis2_ref):
    deg = d0_ref[...] + d1_ref[...] + 1.0
    dis = jnp.where(deg > 0, lax.rsqrt(jnp.maximum(deg, 1e-12)), 0.0)
    dis_ref[...] = dis
    dis2_ref[...] = dis * dis


def _dis(d0, d1):
    return pl.pallas_call(
        _dis_body,
        out_shape=(jax.ShapeDtypeStruct((80, 125), jnp.float32),
                   jax.ShapeDtypeStruct((80, 125), jnp.float32)),
    )(d0, d1)


def _comb_mm_body(p0_ref, p1_ref, hw_ref, dis2_ref, b_ref, w_ref, o_ref):
    h = p0_ref[...] + p1_ref[...] + dis2_ref[...] * hw_ref[...] + b_ref[...]
    h = jnp.maximum(h, 0.0)
    o_ref[...] = jnp.dot(h, w_ref[...], preferred_element_type=jnp.float32)


def _comb_mm(p0, p1, hw, dis2, b2, w):
    grid = N // MM_BLK
    return pl.pallas_call(
        _comb_mm_body,
        grid=(grid,),
        in_specs=[
            pl.BlockSpec((MM_BLK, D), lambda i: (i, 0)),
            pl.BlockSpec((MM_BLK, D), lambda i: (i, 0)),
            pl.BlockSpec((MM_BLK, D), lambda i: (i, 0)),
            pl.BlockSpec((MM_BLK, 1), lambda i: (i, 0)),
            pl.BlockSpec((1, D), lambda i: (0, 0)),
            pl.BlockSpec((D, D), lambda i: (0, 0)),
        ],
        out_specs=pl.BlockSpec((MM_BLK, D), lambda i: (i, 0)),
        out_shape=jax.ShapeDtypeStruct((N, D), jnp.float32),
    )(p0, p1, hw, dis2, b2, w)


def _pool_body(p0_ref, p1_ref, hw_ref, dis2_ref, b_ref, wreg_ref, batch_ref,
               zsum_ref, cnt_ref):
    @pl.when(pl.program_id(0) == 0)
    def _():
        zsum_ref[...] = jnp.zeros_like(zsum_ref)
        cnt_ref[...] = jnp.zeros_like(cnt_ref)

    h = p0_ref[...] + p1_ref[...] + dis2_ref[...] * hw_ref[...] + b_ref[...]
    h = jnp.maximum(h, 0.0)
    z = jnp.dot(h, wreg_ref[...], preferred_element_type=jnp.float32)
    gids = lax.broadcasted_iota(jnp.int32, (MM_BLK, B), 1)
    oh = (batch_ref[...] == gids).astype(jnp.float32)
    zsum_ref[...] += lax.dot_general(
        oh, z, (((0,), (0,)), ((), ())), preferred_element_type=jnp.float32)
    cnt_ref[...] += lax.dot_general(
        oh, jnp.ones((MM_BLK, 1), jnp.float32), (((0,), (0,)), ((), ())),
        preferred_element_type=jnp.float32)


def _pool(p0, p1, hw, dis2, b2, wreg, batch2):
    grid = N // MM_BLK
    return pl.pallas_call(
        _pool_body,
        grid=(grid,),
        in_specs=[
            pl.BlockSpec((MM_BLK, D), lambda i: (i, 0)),
            pl.BlockSpec((MM_BLK, D), lambda i: (i, 0)),
            pl.BlockSpec((MM_BLK, D), lambda i: (i, 0)),
            pl.BlockSpec((MM_BLK, 1), lambda i: (i, 0)),
            pl.BlockSpec((1, D), lambda i: (0, 0)),
            pl.BlockSpec((D, 1), lambda i: (0, 0)),
            pl.BlockSpec((MM_BLK, 1), lambda i: (i, 0)),
        ],
        out_specs=(pl.BlockSpec((B, 1), lambda i: (0, 0)),
                   pl.BlockSpec((B, 1), lambda i: (0, 0))),
        out_shape=(jax.ShapeDtypeStruct((B, 1), jnp.float32),
                   jax.ShapeDtypeStruct((B, 1), jnp.float32)),
    )(p0, p1, hw, dis2, b2, wreg, batch2)


def _worker_id():
    return lax.axis_index("s") * NC + lax.axis_index("c")


def _zero_vmem(buf, nrows):
    def body(i, _):
        r = i // (D // 16)
        l = (i % (D // 16)) * 16
        buf[r, pl.ds(l, 16)] = jnp.zeros((16,), jnp.float32)
        return 0
    lax.fori_loop(0, nrows * (D // 16), body, 0)


def _deg_body(ew_hbm, col_hbm, out_hbm, ew_v, col_v, zbuf, deg_sh):
    c = lax.axis_index("c")
    s = lax.axis_index("s")
    wid = _worker_id()
    pltpu.sync_copy(ew_hbm.at[pl.ds(wid * EPT, EPT)], ew_v)
    pltpu.sync_copy(col_hbm.at[wid], col_v)

    def zb(i, _):
        zbuf[pl.ds(i * 16, 16)] = jnp.zeros((16,), jnp.float32)
        return 0
    lax.fori_loop(0, 2000 // 16, zb, 0)

    @pl.when(s < 5)
    def _():
        pltpu.sync_copy(zbuf, deg_sh.at[pl.ds(s * 2000, 2000)])
    plsc.subcore_barrier()

    def chunk(j, _):
        pltpu.sync_copy(ew_v.at[pl.ds(j * K, K)],
                        deg_sh.at[col_v.at[j]], add=True)
        return 0
    lax.fori_loop(0, NCHUNK, chunk, 0)
    plsc.subcore_barrier()

    @pl.when(s < 5)
    def _():
        pltpu.sync_copy(deg_sh.at[pl.ds(s * 2000, 2000)], zbuf)
        pltpu.sync_copy(zbuf, out_hbm.at[pl.ds(c * N + s * 2000, 2000)])


_deg_kernel = functools.partial(
    pl.kernel,
    out_type=jax.ShapeDtypeStruct((NC * N,), jnp.float32),
    mesh=_MESH,
    compiler_params=_SC_PARAMS,
    scratch_types=[
        pltpu.MemorySpace.VMEM((EPT,), jnp.float32),
        pltpu.MemorySpace.VMEM((NCHUNK, K), jnp.int32),
        pltpu.MemorySpace.VMEM((2000,), jnp.float32),
        pltpu.MemorySpace.VMEM_SHARED((N,), jnp.float32),
    ],
)(_deg_body)


def _norm_body(dis_hbm, row_hbm, col_hbm, ew_hbm, out_hbm,
               dis_v, row_v, col_v, ew_v, norm_v):
    wid = _worker_id()
    base = wid * EPT
    pltpu.sync_copy(dis_hbm, dis_v)
    pltpu.sync_copy(row_hbm.at[pl.ds(base, EPT)], row_v)
    pltpu.sync_copy(col_hbm.at[pl.ds(base, EPT)], col_v)
    pltpu.sync_copy(ew_hbm.at[pl.ds(base, EPT)], ew_v)

    def body(i, _):
        off = i * 16
        r16 = row_v[pl.ds(off, 16)]
        c16 = col_v[pl.ds(off, 16)]
        e16 = ew_v[pl.ds(off, 16)]
        dr = plsc.load_gather(dis_v, [r16])
        dc = plsc.load_gather(dis_v, [c16])
        norm_v[pl.ds(off, 16)] = dr * e16 * dc
        return 0
    lax.fori_loop(0, EPT // 16, body, 0)
    pltpu.sync_copy(norm_v, out_hbm.at[pl.ds(base, EPT)])


_norm_kernel = functools.partial(
    pl.kernel,
    out_type=jax.ShapeDtypeStruct((E,), jnp.float32),
    mesh=_MESH,
    compiler_params=_SC_PARAMS,
    scratch_types=[
        pltpu.MemorySpace.VMEM((N,), jnp.float32),
        pltpu.MemorySpace.VMEM((EPT,), jnp.int32),
        pltpu.MemorySpace.VMEM((EPT,), jnp.int32),
        pltpu.MemorySpace.VMEM((EPT,), jnp.float32),
        pltpu.MemorySpace.VMEM((EPT,), jnp.float32),
    ],
)(_norm_body)


def _msg_body(hw_hbm, row_hbm, col_hbm, norm_hbm, out_hbm,
              row_v, col_v, norm_v, gbuf0, gbuf1, sg0, sg1, ss0, ss1, acc_sh):
    c = lax.axis_index("c")
    s = lax.axis_index("s")
    wid = _worker_id()
    pltpu.sync_copy(row_hbm.at[wid], row_v)
    pltpu.sync_copy(col_hbm.at[wid], col_v)
    pltpu.sync_copy(norm_hbm.at[wid], norm_v)

    _zero_vmem(gbuf0, K)

    @pl.when(s < NT_IO)
    def _():
        for t in range(ROWS_PT // K):
            pltpu.sync_copy(
                gbuf0, acc_sh.at[pl.ds(s * ROWS_PT + t * K, K), :])
        rem = ROWS_PT - (ROWS_PT // K) * K
        pltpu.sync_copy(
            gbuf0.at[pl.ds(0, rem), :],
            acc_sh.at[pl.ds(s * ROWS_PT + (ROWS_PT // K) * K, rem), :])
    plsc.subcore_barrier()

    def scale(gbuf, j):
        def group(g, _):
            nv = norm_v[j, pl.ds(g * 16, 16)]
            for e16 in range(16):
                sc = nv[e16]
                e = g * 16 + e16
                for v in range(D // 16):
                    gbuf[e, pl.ds(v * 16, 16)] = (
                        gbuf[e, pl.ds(v * 16, 16)] * sc)
            return 0
        lax.fori_loop(0, K // 16, group, 0)

    def gather(j, gbuf, sem):
        pltpu.make_async_copy(hw_hbm.at[row_v.at[j]], gbuf, sem).start()

    def gather_wait(j, gbuf, sem):
        pltpu.make_async_copy(hw_hbm.at[row_v.at[j]], gbuf, sem).wait()

    def scatter(j, gbuf, sem):
        pltpu.make_async_copy(
            gbuf, acc_sh.at[col_v.at[j]], sem).start(add=True)

    def scatter_wait(j, gbuf, sem):
        pltpu.make_async_copy(gbuf, acc_sh.at[col_v.at[j]], sem).wait()

    gather(0, gbuf0, sg0)
    gather(1, gbuf1, sg1)

    def pair(i, _):
        j0 = 2 * i
        j1 = j0 + 1
        gather_wait(j0, gbuf0, sg0)
        scale(gbuf0, j0)
        scatter(j0, gbuf0, ss0)
        gather_wait(j1, gbuf1, sg1)
        scale(gbuf1, j1)
        scatter(j1, gbuf1, ss1)
        scatter_wait(j0, gbuf0, ss0)
        gather(j0 + 2, gbuf0, sg0)

        @pl.when(j1 + 2 < NCHUNK)
        def _():
            scatter_wait(j1, gbuf1, ss1)
            gather(j1 + 2, gbuf1, sg1)
        return 0
    lax.fori_loop(0, (NCHUNK - 1) // 2, pair, 0)

    jt = NCHUNK - 1
    gather_wait(jt, gbuf0, sg0)
    scale(gbuf0, jt)
    scatter(jt, gbuf0, ss0)
    scatter_wait(jt, gbuf0, ss0)
    scatter_wait(jt - 1, gbuf1, ss1)
    plsc.subcore_barrier()

    @pl.when(s < NT_IO)
    def _():
        nfull = ROWS_PT // K
        rem = ROWS_PT - nfull * K
        for t in range(nfull):
            r0 = s * ROWS_PT + t * K
            pltpu.sync_copy(acc_sh.at[pl.ds(r0, K), :], gbuf0)
            pltpu.sync_copy(gbuf0, out_hbm.at[c, pl.ds(r0, K), :])
        r0 = s * ROWS_PT + nfull * K
        pltpu.sync_copy(acc_sh.at[pl.ds(r0, rem), :], gbuf0.at[pl.ds(0, rem), :])
        pltpu.sync_copy(gbuf0.at[pl.ds(0, rem), :],
                        out_hbm.at[c, pl.ds(r0, rem), :])


_msg_kernel = functools.partial(
    pl.kernel,
    out_type=jax.ShapeDtypeStruct((NC, N, D), jnp.float32),
    mesh=_MESH,
    compiler_params=_SC_PARAMS,
    scratch_types=[
        pltpu.MemorySpace.VMEM((NCHUNK, K), jnp.int32),
        pltpu.MemorySpace.VMEM((NCHUNK, K), jnp.int32),
        pltpu.MemorySpace.VMEM((NCHUNK, K), jnp.float32),
        pltpu.MemorySpace.VMEM((K, D), jnp.float32),
        pltpu.MemorySpace.VMEM((K, D), jnp.float32),
        pltpu.SemaphoreType.DMA,
        pltpu.SemaphoreType.DMA,
        pltpu.SemaphoreType.DMA,
        pltpu.SemaphoreType.DMA,
        pltpu.MemorySpace.VMEM_SHARED((N, D), jnp.float32),
    ],
)(_msg_body)


def kernel(x, edge_index, edge_attr, batch, W_node, b_node, W_edge, b_edge,
           W_g0, b_g0, W_g1, b_g1, W_g2, b_g2, W_reg, b_reg):
    row = edge_index[0]
    col = edge_index[1]
    row3 = row.reshape(NW, NCHUNK, K)
    col3 = col.reshape(NW, NCHUNK, K)

    wbar = jnp.mean(W_edge, axis=1)
    bbar = jnp.mean(b_edge)
    wk = jnp.kron(jnp.eye(32, dtype=jnp.float32), wbar[:, None])

    h0 = _mm(x, W_node, b_node[None, :])

    attr_r = edge_attr.reshape(E // 32, 128)
    ew_mat = _mm(attr_r, wk, jnp.full((1, 32), bbar, jnp.float32))
    ew = ew_mat.reshape(E)

    deg_parts = _deg_kernel(ew, col3)
    dis_m, dis2_m = _dis(deg_parts[:N].reshape(80, 125),
                         deg_parts[N:].reshape(80, 125))
    dis = dis_m.reshape(N)
    dis2 = dis2_m.reshape(N, 1)

    norm = _norm_kernel(dis, row, col, ew)
    norm3 = norm.reshape(NW, NCHUNK, K)

    hw = _mm(h0, W_g0, jnp.zeros((1, D), jnp.float32))
    parts = _msg_kernel(hw, row3, col3, norm3)
    hw1 = _comb_mm(parts[0], parts[1], hw, dis2, b_g0[None, :], W_g1)
    parts1 = _msg_kernel(hw1, row3, col3, norm3)
    hw2 = _comb_mm(parts1[0], parts1[1], hw1, dis2, b_g1[None, :], W_g2)
    parts2 = _msg_kernel(hw2, row3, col3, norm3)

    zsum, cnt = _pool(parts2[0], parts2[1], hw2, dis2, b_g2[None, :],
                      W_reg, batch.reshape(N, 1))
    return zsum / jnp.maximum(cnt, 1.0) + b_reg


# speedup vs baseline: 16.7743x; 1.1641x over previous
import functools

import jax
import jax.numpy as jnp
from jax import lax
from jax.experimental import pallas as pl
from jax.experimental.pallas import tpu as pltpu
from jax.experimental.pallas import tpu_sc as plsc

N = 10000
E = 320000
D = 128
B = 8

NC = 2
NS = 16
NW = NC * NS
EPT = E // NW
K = 80
NCHUNK = EPT // K

NT_IO = 10
ROWS_PT = N // NT_IO

_MESH = plsc.VectorSubcoreMesh(
    core_axis_name="c", subcore_axis_name="s", num_cores=NC, num_subcores=NS)
_SC_PARAMS = pltpu.CompilerParams(
    use_tc_tiling_on_sc=False, needs_layout_passes=False)

MM_BLK = 2000


def _mm_body(a_ref, w_ref, b_ref, o_ref):
    o_ref[...] = (
        jnp.dot(a_ref[...], w_ref[...], preferred_element_type=jnp.float32)
        + b_ref[...])


def _mm(a, w, b2):
    n, k = a.shape
    m = w.shape[1]
    grid = n // MM_BLK
    return pl.pallas_call(
        _mm_body,
        grid=(grid,),
        in_specs=[
            pl.BlockSpec((MM_BLK, k), lambda i: (i, 0)),
            pl.BlockSpec((k, m), lambda i: (0, 0)),
            pl.BlockSpec((1, m), lambda i: (0, 0)),
        ],
        out_specs=pl.BlockSpec((MM_BLK, m), lambda i: (i, 0)),
        out_shape=jax.ShapeDtypeStruct((n, m), jnp.float32),
    )(a, w, b2)


def _dis_body(d0_ref, d1_ref, dis_ref, dis2_ref):
    deg = d0_ref[...] + d1_ref[...] + 1.0
    dis = jnp.where(deg > 0, lax.rsqrt(jnp.maximum(deg, 1e-12)), 0.0)
    dis_ref[...] = dis
    dis2_ref[...] = dis * dis


def _dis(d0, d1):
    return pl.pallas_call(
        _dis_body,
        out_shape=(jax.ShapeDtypeStruct((80, 125), jnp.float32),
                   jax.ShapeDtypeStruct((80, 125), jnp.float32)),
    )(d0, d1)


def _comb_mm_body(p0_ref, p1_ref, hw_ref, dis2_ref, b_ref, w_ref, o_ref):
    h = p0_ref[...] + p1_ref[...] + dis2_ref[...] * hw_ref[...] + b_ref[...]
    h = jnp.maximum(h, 0.0)
    o_ref[...] = jnp.dot(h, w_ref[...], preferred_element_type=jnp.float32)


def _comb_mm(p0, p1, hw, dis2, b2, w):
    grid = N // MM_BLK
    return pl.pallas_call(
        _comb_mm_body,
        grid=(grid,),
        in_specs=[
            pl.BlockSpec((MM_BLK, D), lambda i: (i, 0)),
            pl.BlockSpec((MM_BLK, D), lambda i: (i, 0)),
            pl.BlockSpec((MM_BLK, D), lambda i: (i, 0)),
            pl.BlockSpec((MM_BLK, 1), lambda i: (i, 0)),
            pl.BlockSpec((1, D), lambda i: (0, 0)),
            pl.BlockSpec((D, D), lambda i: (0, 0)),
        ],
        out_specs=pl.BlockSpec((MM_BLK, D), lambda i: (i, 0)),
        out_shape=jax.ShapeDtypeStruct((N, D), jnp.float32),
    )(p0, p1, hw, dis2, b2, w)


def _pool_body(p0_ref, p1_ref, hw_ref, dis2_ref, b_ref, batch_ref,
               hsum_ref, cnt_ref):
    @pl.when(pl.program_id(0) == 0)
    def _():
        hsum_ref[...] = jnp.zeros_like(hsum_ref)
        cnt_ref[...] = jnp.zeros_like(cnt_ref)

    h = p0_ref[...] + p1_ref[...] + dis2_ref[...] * hw_ref[...] + b_ref[...]
    h = jnp.maximum(h, 0.0)
    gids = lax.broadcasted_iota(jnp.int32, (MM_BLK, B), 1)
    oh = (batch_ref[...] == gids).astype(jnp.float32)
    hsum_ref[...] += lax.dot_general(
        oh, h, (((0,), (0,)), ((), ())), preferred_element_type=jnp.float32)
    cnt_ref[...] += lax.dot_general(
        oh, jnp.ones((MM_BLK, 1), jnp.float32), (((0,), (0,)), ((), ())),
        preferred_element_type=jnp.float32)


def _pool(p0, p1, hw, dis2, b2, batch2):
    grid = N // MM_BLK
    return pl.pallas_call(
        _pool_body,
        grid=(grid,),
        in_specs=[
            pl.BlockSpec((MM_BLK, D), lambda i: (i, 0)),
            pl.BlockSpec((MM_BLK, D), lambda i: (i, 0)),
            pl.BlockSpec((MM_BLK, D), lambda i: (i, 0)),
            pl.BlockSpec((MM_BLK, 1), lambda i: (i, 0)),
            pl.BlockSpec((1, D), lambda i: (0, 0)),
            pl.BlockSpec((MM_BLK, 1), lambda i: (i, 0)),
        ],
        out_specs=(pl.BlockSpec((B, D), lambda i: (0, 0)),
                   pl.BlockSpec((B, 1), lambda i: (0, 0))),
        out_shape=(jax.ShapeDtypeStruct((B, D), jnp.float32),
                   jax.ShapeDtypeStruct((B, 1), jnp.float32)),
    )(p0, p1, hw, dis2, b2, batch2)


def _worker_id():
    return lax.axis_index("s") * NC + lax.axis_index("c")


def _zero_vmem(buf, nrows):
    def body(i, _):
        r = i // (D // 16)
        l = (i % (D // 16)) * 16
        buf[r, pl.ds(l, 16)] = jnp.zeros((16,), jnp.float32)
        return 0
    lax.fori_loop(0, nrows * (D // 16), body, 0)


def _deg_body(ew_hbm, col_hbm, out_hbm, ew_v, col_v, zbuf, deg_sh):
    c = lax.axis_index("c")
    s = lax.axis_index("s")
    wid = _worker_id()
    pltpu.sync_copy(ew_hbm.at[pl.ds(wid * EPT, EPT)], ew_v)
    pltpu.sync_copy(col_hbm.at[wid], col_v)

    def zb(i, _):
        zbuf[pl.ds(i * 16, 16)] = jnp.zeros((16,), jnp.float32)
        return 0
    lax.fori_loop(0, 2000 // 16, zb, 0)

    @pl.when(s < 5)
    def _():
        pltpu.sync_copy(zbuf, deg_sh.at[pl.ds(s * 2000, 2000)])
    plsc.subcore_barrier()

    def chunk(j, _):
        pltpu.sync_copy(ew_v.at[pl.ds(j * K, K)],
                        deg_sh.at[col_v.at[j]], add=True)
        return 0
    lax.fori_loop(0, NCHUNK, chunk, 0)
    plsc.subcore_barrier()

    @pl.when(s < 5)
    def _():
        pltpu.sync_copy(deg_sh.at[pl.ds(s * 2000, 2000)], zbuf)
        pltpu.sync_copy(zbuf, out_hbm.at[pl.ds(c * N + s * 2000, 2000)])


_deg_kernel = functools.partial(
    pl.kernel,
    out_type=jax.ShapeDtypeStruct((NC * N,), jnp.float32),
    mesh=_MESH,
    compiler_params=_SC_PARAMS,
    scratch_types=[
        pltpu.MemorySpace.VMEM((EPT,), jnp.float32),
        pltpu.MemorySpace.VMEM((NCHUNK, K), jnp.int32),
        pltpu.MemorySpace.VMEM((2000,), jnp.float32),
        pltpu.MemorySpace.VMEM_SHARED((N,), jnp.float32),
    ],
)(_deg_body)


def _norm_body(dis_hbm, row_hbm, col_hbm, ew_hbm, out_hbm,
               dis_v, row_v, col_v, ew_v, norm_v):
    wid = _worker_id()
    base = wid * EPT
    pltpu.sync_copy(dis_hbm, dis_v)
    pltpu.sync_copy(row_hbm.at[pl.ds(base, EPT)], row_v)
    pltpu.sync_copy(col_hbm.at[pl.ds(base, EPT)], col_v)
    pltpu.sync_copy(ew_hbm.at[pl.ds(base, EPT)], ew_v)

    def body(i, _):
        off = i * 16
        r16 = row_v[pl.ds(off, 16)]
        c16 = col_v[pl.ds(off, 16)]
        e16 = ew_v[pl.ds(off, 16)]
        dr = plsc.load_gather(dis_v, [r16])
        dc = plsc.load_gather(dis_v, [c16])
        norm_v[pl.ds(off, 16)] = dr * e16 * dc
        return 0
    lax.fori_loop(0, EPT // 16, body, 0)
    pltpu.sync_copy(norm_v, out_hbm.at[pl.ds(base, EPT)])


_norm_kernel = functools.partial(
    pl.kernel,
    out_type=jax.ShapeDtypeStruct((E,), jnp.float32),
    mesh=_MESH,
    compiler_params=_SC_PARAMS,
    scratch_types=[
        pltpu.MemorySpace.VMEM((N,), jnp.float32),
        pltpu.MemorySpace.VMEM((EPT,), jnp.int32),
        pltpu.MemorySpace.VMEM((EPT,), jnp.int32),
        pltpu.MemorySpace.VMEM((EPT,), jnp.float32),
        pltpu.MemorySpace.VMEM((EPT,), jnp.float32),
    ],
)(_norm_body)


def _msg_body(hw_hbm, row_hbm, col_hbm, norm_hbm, out_hbm,
              row_c, col_c, norm_c, gbuf0, gbuf1, gbuf2,
              sg0, sg1, sg2, ss0, ss1, ss2,
              si0, si1, si2, si3, si4, si5, acc_sh):
    c = lax.axis_index("c")
    s = lax.axis_index("s")
    wid = _worker_id()
    bufs = (gbuf0, gbuf1, gbuf2)
    gsems = (sg0, sg1, sg2)
    ssems = (ss0, ss1, ss2)
    isems = (si0, si1, si2, si3, si4, si5)

    _zero_vmem(gbuf0, K)

    @pl.when(s < NT_IO)
    def _():
        for t in range(ROWS_PT // K):
            pltpu.sync_copy(
                gbuf0, acc_sh.at[pl.ds(s * ROWS_PT + t * K, K), :])
        rem = ROWS_PT - (ROWS_PT // K) * K
        pltpu.sync_copy(
            gbuf0.at[pl.ds(0, rem), :],
            acc_sh.at[pl.ds(s * ROWS_PT + (ROWS_PT // K) * K, rem), :])
    plsc.subcore_barrier()

    def scale(gbuf, sl):
        def group(g, _):
            nv = norm_c[sl, pl.ds(g * 16, 16)]

            def edge(r, _):
                e = g * 16 + r
                bc = lax.gather(
                    nv, jnp.full((16, 1), r, jnp.int32),
                    dimension_numbers=lax.GatherDimensionNumbers(
                        offset_dims=(), collapsed_slice_dims=(0,),
                        start_index_map=(0,)),
                    slice_sizes=(1,),
                    mode=lax.GatherScatterMode.PROMISE_IN_BOUNDS)
                for v in range(D // 16):
                    gbuf[e, pl.ds(v * 16, 16)] = (
                        gbuf[e, pl.ds(v * 16, 16)] * bc)
                return 0
            lax.fori_loop(0, 16, edge, 0)
            return 0
        lax.fori_loop(0, K // 16, group, 0)

    def idx_start(j, sl):
        pltpu.make_async_copy(row_hbm.at[wid, j], row_c.at[sl],
                              isems[sl]).start()
        pltpu.make_async_copy(col_hbm.at[wid, j], col_c.at[sl],
                              isems[sl]).start()
        pltpu.make_async_copy(norm_hbm.at[wid, j], norm_c.at[sl],
                              isems[sl]).start()

    def idx_wait(j, sl):
        pltpu.make_async_copy(row_hbm.at[wid, j], row_c.at[sl],
                              isems[sl]).wait()
        pltpu.make_async_copy(col_hbm.at[wid, j], col_c.at[sl],
                              isems[sl]).wait()
        pltpu.make_async_copy(norm_hbm.at[wid, j], norm_c.at[sl],
                              isems[sl]).wait()

    def gather_start(b, sl):
        pltpu.make_async_copy(hw_hbm.at[row_c.at[sl]], bufs[b],
                              gsems[b]).start()

    def gather_wait(b, sl):
        pltpu.make_async_copy(hw_hbm.at[row_c.at[sl]], bufs[b],
                              gsems[b]).wait()

    def scatter_start(b, sl):
        pltpu.make_async_copy(bufs[b], acc_sh.at[col_c.at[sl]],
                              ssems[b]).start(add=True)

    def scatter_wait(b, sl):
        pltpu.make_async_copy(bufs[b], acc_sh.at[col_c.at[sl]],
                              ssems[b]).wait()

    for t in range(4):
        idx_start(t, t)
    idx_wait(0, 0)
    gather_start(0, 0)

    def step(j, k):
        @pl.when(j - 2 >= 0)
        def _():
            scatter_wait((k + 1) % 3, (k + 4) % 6)
        @pl.when(j + 4 < NCHUNK)
        def _():
            idx_start(j + 4, (k + 4) % 6)
        @pl.when(j + 1 < NCHUNK)
        def _():
            idx_wait(j + 1, (k + 1) % 6)
            gather_start((k + 1) % 3, (k + 1) % 6)
        gather_wait(k % 3, k)
        scale(bufs[k % 3], k)
        scatter_start(k % 3, k)

    def six(i, _):
        for k in range(6):
            step(6 * i + k, k)
        return 0
    lax.fori_loop(0, NCHUNK // 6, six, 0)
    for k in range(NCHUNK - (NCHUNK // 6) * 6):
        step((NCHUNK // 6) * 6 + k, k)
    scatter_wait((NCHUNK - 2) % 3, (NCHUNK - 2) % 6)
    scatter_wait((NCHUNK - 1) % 3, (NCHUNK - 1) % 6)
    plsc.subcore_barrier()

    @pl.when(s < NT_IO)
    def _():
        nfull = ROWS_PT // K
        rem = ROWS_PT - nfull * K
        for t in range(nfull):
            r0 = s * ROWS_PT + t * K
            pltpu.sync_copy(acc_sh.at[pl.ds(r0, K), :], gbuf0)
            pltpu.sync_copy(gbuf0, out_hbm.at[c, pl.ds(r0, K), :])
        r0 = s * ROWS_PT + nfull * K
        pltpu.sync_copy(acc_sh.at[pl.ds(r0, rem), :], gbuf0.at[pl.ds(0, rem), :])
        pltpu.sync_copy(gbuf0.at[pl.ds(0, rem), :],
                        out_hbm.at[c, pl.ds(r0, rem), :])


_msg_kernel = functools.partial(
    pl.kernel,
    out_type=jax.ShapeDtypeStruct((NC, N, D), jnp.float32),
    mesh=_MESH,
    compiler_params=_SC_PARAMS,
    scratch_types=[
        pltpu.MemorySpace.VMEM((6, K), jnp.int32),
        pltpu.MemorySpace.VMEM((6, K), jnp.int32),
        pltpu.MemorySpace.VMEM((6, K), jnp.float32),
        pltpu.MemorySpace.VMEM((K, D), jnp.float32),
        pltpu.MemorySpace.VMEM((K, D), jnp.float32),
        pltpu.MemorySpace.VMEM((K, D), jnp.float32),
    ] + [pltpu.SemaphoreType.DMA] * 12 + [
        pltpu.MemorySpace.VMEM_SHARED((N, D), jnp.float32),
    ],
)(_msg_body)


def kernel(x, edge_index, edge_attr, batch, W_node, b_node, W_edge, b_edge,
           W_g0, b_g0, W_g1, b_g1, W_g2, b_g2, W_reg, b_reg):
    row = edge_index[0]
    col = edge_index[1]
    row3 = row.reshape(NW, NCHUNK, K)
    col3 = col.reshape(NW, NCHUNK, K)

    wbar = jnp.mean(W_edge, axis=1)
    bbar = jnp.mean(b_edge)
    wk = jnp.kron(jnp.eye(32, dtype=jnp.float32), wbar[:, None])

    h0 = _mm(x, W_node, b_node[None, :])

    attr_r = edge_attr.reshape(E // 32, 128)
    ew_mat = _mm(attr_r, wk, jnp.full((1, 32), bbar, jnp.float32))
    ew = ew_mat.reshape(E)

    deg_parts = _deg_kernel(ew, col3)
    dis_m, dis2_m = _dis(deg_parts[:N].reshape(80, 125),
                         deg_parts[N:].reshape(80, 125))
    dis = dis_m.reshape(N)
    dis2 = dis2_m.reshape(N, 1)

    norm = _norm_kernel(dis, row, col, ew)
    norm3 = norm.reshape(NW, NCHUNK, K)

    hw = _mm(h0, W_g0, jnp.zeros((1, D), jnp.float32))
    parts = _msg_kernel(hw, row3, col3, norm3)
    hw1 = _comb_mm(parts[0], parts[1], hw, dis2, b_g0[None, :], W_g1)
    parts1 = _msg_kernel(hw1, row3, col3, norm3)
    hw2 = _comb_mm(parts1[0], parts1[1], hw1, dis2, b_g1[None, :], W_g2)
    parts2 = _msg_kernel(hw2, row3, col3, norm3)

    hsum, cnt = _pool(parts2[0], parts2[1], hw2, dis2, b_g2[None, :],
                      batch.reshape(N, 1))
    pooled = hsum / jnp.maximum(cnt, 1.0)
    return pooled @ W_reg + b_reg
